# baseline (speedup 1.0000x reference)
import functools

import jax
import jax.numpy as jnp
from jax import lax
from jax.experimental import pallas as pl
from jax.experimental.pallas import tpu as pltpu
from jax.experimental.pallas import tpu_sc as plsc

N_NODES = 10000
D_FEAT = 128
N_EDGES = 320000
R_DEC = 2.0
T_DEC = 1.0

L = 16
NC, NS = 2, 16
NW = NC * NS
GIDX = 128
EPC = GIDX // 2
NCH = 160
EPW = NCH * EPC
E_PAD = NW * EPW
GRP = 16
NBUF = 4


def _lp_body(h_hbm, idx_hbm, out_hbm, idx_v, rows0_v, rows1_v, rows2_v,
             rows3_v, sq_v, scr_v, sem0, sem1, sem2, sem3):
    wid = lax.axis_index("s") * NC + lax.axis_index("c")
    ebase = wid * EPW

    pltpu.sync_copy(idx_hbm.at[wid], idx_v)

    lanes = lax.iota(jnp.int32, L)

    def start(c, rows, sem):
        pltpu.async_copy(h_hbm.at[idx_v.at[c]], rows, sem)

    def drain(rows, sem):
        pltpu.make_async_copy(h_hbm.at[idx_v.at[0]], rows, sem).wait()

    def compute(c, rows):
        def group_body(g, carry):
            for l in range(GRP):
                m = g * GRP + l
                acc = jnp.zeros((L,), jnp.float32)
                for d in range(D_FEAT // (2 * L)):
                    a = plsc.bitcast(rows[2 * m, pl.ds(d * L, L)],
                                     jnp.bfloat16)
                    b = plsc.bitcast(rows[2 * m + 1, pl.ds(d * L, L)],
                                     jnp.bfloat16)
                    t0, t1 = plsc.unpack(a - b,
                                         format=plsc.PackFormat.INTERLEAVED)
                    acc = acc + t0 * t0 + t1 * t1
                scr_v[l] = acc
            s = jnp.zeros((L,), jnp.float32)
            for k in range(L):
                s = s + plsc.load_gather(
                    scr_v, [lanes, jnp.full((L,), k, jnp.int32)])
            pos = jnp.full((L,), c * EPC + g * GRP, jnp.int32) + lanes
            plsc.store_scatter(sq_v, [pos], s)
            return carry

        lax.fori_loop(0, EPC // GRP, group_body, 0)

    bufs = [(rows0_v, sem0), (rows1_v, sem1), (rows2_v, sem2), (rows3_v, sem3)]
    for i in range(NBUF):
        start(i, *bufs[i])

    def quad_body(p, carry):
        c0 = NBUF * p
        for i in range(NBUF):
            rows, sem = bufs[i]
            drain(rows, sem)
            compute(c0 + i, rows)

            @pl.when(c0 + NBUF + i < NCH)
            def _():
                start(c0 + NBUF + i, rows, sem)

        return carry

    lax.fori_loop(0, NCH // NBUF, quad_body, 0)

    inv_t = 1.0 / T_DEC

    def sig_body(t, carry):
        v = sq_v[pl.ds(t * L, L)]
        sq_v[pl.ds(t * L, L)] = 1.0 / (jnp.exp((v - R_DEC) * inv_t) + 1.0)
        return carry

    lax.fori_loop(0, EPW // L, sig_body, 0)
    pltpu.sync_copy(sq_v, out_hbm.at[pl.ds(ebase, EPW)])


@jax.jit
def _lp_call(h, idx2d):
    mesh = plsc.VectorSubcoreMesh(core_axis_name="c", subcore_axis_name="s")
    fn = functools.partial(
        pl.kernel,
        out_type=jax.ShapeDtypeStruct((E_PAD,), jnp.float32),
        mesh=mesh,
        compiler_params=pltpu.CompilerParams(needs_layout_passes=False,
                                             use_tc_tiling_on_sc=False),
        scratch_types=[
            pltpu.VMEM((NCH, GIDX), jnp.int32),
            pltpu.VMEM((GIDX, D_FEAT // 2), jnp.int32),
            pltpu.VMEM((GIDX, D_FEAT // 2), jnp.int32),
            pltpu.VMEM((GIDX, D_FEAT // 2), jnp.int32),
            pltpu.VMEM((GIDX, D_FEAT // 2), jnp.int32),
            pltpu.VMEM((EPW,), jnp.float32),
            pltpu.VMEM((L, L), jnp.float32),
            pltpu.SemaphoreType.DMA,
            pltpu.SemaphoreType.DMA,
            pltpu.SemaphoreType.DMA,
            pltpu.SemaphoreType.DMA,
        ],
    )(_lp_body)
    return fn(h, idx2d)


def kernel(h, idx):
    hp = lax.bitcast_convert_type(
        h.astype(jnp.bfloat16).reshape(N_NODES, D_FEAT // 2, 2), jnp.int32)
    idx_flat = idx.astype(jnp.int32).reshape(-1)
    idx_pad = jnp.concatenate(
        [idx_flat, jnp.zeros((2 * (E_PAD - N_EDGES),), jnp.int32)])
    idx2d = idx_pad.reshape(NW, NCH, GIDX)
    return _lp_call(hp, idx2d)[:N_EDGES]


# speedup vs baseline: 3.2398x; 3.2398x over previous
import functools

import jax
import jax.numpy as jnp
from jax import lax
from jax.experimental import pallas as pl
from jax.experimental.pallas import tpu as pltpu
from jax.experimental.pallas import tpu_sc as plsc

N_NODES = 10000
D_FEAT = 128
N_EDGES = 320000
R_DEC = 2.0
T_DEC = 1.0

L = 16
NC, NS = 2, 16
NW = NC * NS
EPW = N_EDGES // NW
EPC = 80
NCH = EPW // EPC
GRP = 16
NBUF = 8


def _lp_body(hp_hbm, hn_hbm, idxi_hbm, idxo_hbm, out_hbm,
             idxi_v, idxo_v, rows_v, sq_v, scr_v, sems):
    wid = lax.axis_index("s") * NC + lax.axis_index("c")
    ebase = wid * EPW

    pltpu.sync_copy(idxi_hbm.at[wid], idxi_v)
    pltpu.sync_copy(idxo_hbm.at[wid], idxo_v)

    lanes = lax.iota(jnp.int32, L)

    def start_a(c, b):
        pltpu.async_copy(hp_hbm.at[idxi_v.at[c]], rows_v.at[b], sems.at[b])

    def start_b(c, b):
        pltpu.async_copy(hn_hbm.at[idxo_v.at[c]], rows_v.at[b], sems.at[b],
                         add=True)

    def drain(b):
        pltpu.make_async_copy(
            hp_hbm.at[idxi_v.at[0]], rows_v.at[b], sems.at[b]).wait()

    def compute(c, b):
        rows = rows_v.at[b]

        def group_body(g, carry):
            for l in range(GRP):
                m = g * GRP + l
                acc = jnp.zeros((L,), jnp.float32)
                for d in range(D_FEAT // (2 * L)):
                    t0, t1 = plsc.unpack(rows[m, pl.ds(2 * d * L, 2 * L)],
                                         format=plsc.PackFormat.INTERLEAVED)
                    acc = acc + t0 * t0 + t1 * t1
                scr_v[l] = acc
            s = jnp.zeros((L,), jnp.float32)
            for k in range(L):
                s = s + plsc.load_gather(
                    scr_v, [lanes, jnp.full((L,), k, jnp.int32)])
            pos = jnp.full((L,), c * EPC + g * GRP, jnp.int32) + lanes
            plsc.store_scatter(sq_v, [pos], s)
            return carry

        lax.fori_loop(0, EPC // GRP, group_body, 0)

    half = NBUF // 2
    for i in range(half):
        start_a(i, i)
    for i in range(half):
        drain(i)
        start_b(i, i)
        start_a(half + i, half + i)

    def ring_body(p, carry):
        c0 = NBUF * p
        for i in range(NBUF):
            c = c0 + i
            drain(i)
            compute(c, i)

            @pl.when(c + NBUF < NCH)
            def _():
                start_a(c + NBUF, i)

            j = (i + half) % NBUF

            @pl.when(c + half < NCH)
            def _():
                drain(j)
                start_b(c + half, j)

        return carry

    lax.fori_loop(0, NCH // NBUF, ring_body, 0)
    for i in range(NCH % NBUF):
        c = NBUF * (NCH // NBUF) + i
        if c + half < NCH:
            j = (i + half) % NBUF
            drain(j)
            start_b(c + half, j)
        drain(i)
        compute(c, i)

    inv_t = 1.0 / T_DEC

    def sig_body(t, carry):
        v = sq_v[pl.ds(t * L, L)]
        sq_v[pl.ds(t * L, L)] = 1.0 / (jnp.exp((v - R_DEC) * inv_t) + 1.0)
        return carry

    lax.fori_loop(0, EPW // L, sig_body, 0)
    pltpu.sync_copy(sq_v, out_hbm.at[pl.ds(ebase, EPW)])


@jax.jit
def _lp_call(hp, hn, idxi, idxo):
    mesh = plsc.VectorSubcoreMesh(core_axis_name="c", subcore_axis_name="s")
    fn = functools.partial(
        pl.kernel,
        out_type=jax.ShapeDtypeStruct((N_EDGES,), jnp.float32),
        mesh=mesh,
        compiler_params=pltpu.CompilerParams(needs_layout_passes=False,
                                             use_tc_tiling_on_sc=False),
        scratch_types=[
            pltpu.VMEM((NCH, EPC), jnp.int32),
            pltpu.VMEM((NCH, EPC), jnp.int32),
            pltpu.VMEM((NBUF, EPC, D_FEAT), jnp.bfloat16),
            pltpu.VMEM((EPW,), jnp.float32),
            pltpu.VMEM((L, L), jnp.float32),
            pltpu.SemaphoreType.DMA((NBUF,)),
        ],
    )(_lp_body)
    return fn(hp, hn, idxi, idxo)


def kernel(h, idx):
    hb = h.astype(jnp.bfloat16)
    idxi = idx[:, 0].astype(jnp.int32).reshape(NW, NCH, EPC)
    idxo = idx[:, 1].astype(jnp.int32).reshape(NW, NCH, EPC)
    return _lp_call(hb, -hb, idxi, idxo)


# speedup vs baseline: 3.4066x; 1.0515x over previous
import functools

import jax
import jax.numpy as jnp
from jax import lax
from jax.experimental import pallas as pl
from jax.experimental.pallas import tpu as pltpu
from jax.experimental.pallas import tpu_sc as plsc

N_NODES = 10000
D_FEAT = 128
N_EDGES = 320000
R_DEC = 2.0
T_DEC = 1.0

L = 16
NC, NS = 2, 16
NW = NC * NS
EPW = N_EDGES // NW
EPC = 80
NCH = EPW // EPC
GRP = 16
NBUF = 8


def _lp_body(hp_hbm, hn_hbm, idxi_hbm, idxo_hbm, out_hbm,
             idxi_v, idxo_v, rows_v, sq_v, scr_v, sems):
    wid = lax.axis_index("s") * NC + lax.axis_index("c")
    ebase = wid * EPW

    pltpu.sync_copy(idxi_hbm.at[wid], idxi_v)
    pltpu.sync_copy(idxo_hbm.at[wid], idxo_v)

    lanes = lax.iota(jnp.int32, L)

    def start_a(c, b):
        pltpu.async_copy(hp_hbm.at[idxi_v.at[c]], rows_v.at[b], sems.at[b])

    def start_b(c, b):
        pltpu.async_copy(hn_hbm.at[idxo_v.at[c]], rows_v.at[b], sems.at[b],
                         add=True)

    def drain(b):
        pltpu.make_async_copy(
            hp_hbm.at[idxi_v.at[0]], rows_v.at[b], sems.at[b]).wait()

    def compute(c, b):
        rows = rows_v.at[b]

        def group_body(g, carry):
            for l in range(GRP):
                m = g * GRP + l
                t = rows[m, pl.ds(0, 2 * L)]
                acc = t * t
                for d in range(1, D_FEAT // (2 * L)):
                    t = rows[m, pl.ds(2 * d * L, 2 * L)]
                    acc = acc + t * t
                a0, a1 = plsc.unpack(acc, format=plsc.PackFormat.INTERLEAVED)
                scr_v[l] = a0 + a1
            s = jnp.zeros((L,), jnp.float32)
            for k in range(L):
                s = s + plsc.load_gather(
                    scr_v, [lanes, jnp.full((L,), k, jnp.int32)])
            pos = jnp.full((L,), c * EPC + g * GRP, jnp.int32) + lanes
            plsc.store_scatter(sq_v, [pos], s)
            return carry

        lax.fori_loop(0, EPC // GRP, group_body, 0)

    half = NBUF // 2
    for i in range(half):
        start_a(i, i)
    for i in range(half):
        drain(i)
        start_b(i, i)
        start_a(half + i, half + i)

    def ring_body(p, carry):
        c0 = NBUF * p
        for i in range(NBUF):
            c = c0 + i
            drain(i)
            compute(c, i)

            @pl.when(c + NBUF < NCH)
            def _():
                start_a(c + NBUF, i)

            j = (i + half) % NBUF

            @pl.when(c + half < NCH)
            def _():
                drain(j)
                start_b(c + half, j)

        return carry

    lax.fori_loop(0, NCH // NBUF, ring_body, 0)
    for i in range(NCH % NBUF):
        c = NBUF * (NCH // NBUF) + i
        if c + half < NCH:
            j = (i + half) % NBUF
            drain(j)
            start_b(c + half, j)
        drain(i)
        compute(c, i)

    inv_t = 1.0 / T_DEC

    def sig_body(t, carry):
        v = sq_v[pl.ds(t * L, L)]
        sq_v[pl.ds(t * L, L)] = 1.0 / (jnp.exp((v - R_DEC) * inv_t) + 1.0)
        return carry

    lax.fori_loop(0, EPW // L, sig_body, 0)
    pltpu.sync_copy(sq_v, out_hbm.at[pl.ds(ebase, EPW)])


@jax.jit
def _lp_call(hp, hn, idxi, idxo):
    mesh = plsc.VectorSubcoreMesh(core_axis_name="c", subcore_axis_name="s")
    fn = functools.partial(
        pl.kernel,
        out_type=jax.ShapeDtypeStruct((N_EDGES,), jnp.float32),
        mesh=mesh,
        compiler_params=pltpu.CompilerParams(needs_layout_passes=False,
                                             use_tc_tiling_on_sc=False),
        scratch_types=[
            pltpu.VMEM((NCH, EPC), jnp.int32),
            pltpu.VMEM((NCH, EPC), jnp.int32),
            pltpu.VMEM((NBUF, EPC, D_FEAT), jnp.bfloat16),
            pltpu.VMEM((EPW,), jnp.float32),
            pltpu.VMEM((L, L), jnp.float32),
            pltpu.SemaphoreType.DMA((NBUF,)),
        ],
    )(_lp_body)
    return fn(hp, hn, idxi, idxo)


def kernel(h, idx):
    hb = h.astype(jnp.bfloat16)
    idxi = idx[:, 0].astype(jnp.int32).reshape(NW, NCH, EPC)
    idxo = idx[:, 1].astype(jnp.int32).reshape(NW, NCH, EPC)
    return _lp_call(hb, -hb, idxi, idxo)


# speedup vs baseline: 3.9477x; 1.1589x over previous
import functools

import jax
import jax.numpy as jnp
from jax import lax
from jax.experimental import pallas as pl
from jax.experimental.pallas import tpu as pltpu
from jax.experimental.pallas import tpu_sc as plsc

N_NODES = 10000
D_FEAT = 128
N_EDGES = 320000
R_DEC = 2.0
T_DEC = 1.0

L = 16
NC, NS = 2, 16
NW = NC * NS
EPW = N_EDGES // NW
EPC = 80
NCH = EPW // EPC
GRP = 16
NBUF = 8


def _lp_body(hp_hbm, hn_hbm, idxi_hbm, idxo_hbm, out_hbm,
             idxi_v, idxo_v, rows_v, sq_v, scr_v, sems):
    wid = lax.axis_index("s") * NC + lax.axis_index("c")
    ebase = wid * EPW

    pltpu.sync_copy(idxi_hbm.at[wid], idxi_v)
    pltpu.sync_copy(idxo_hbm.at[wid], idxo_v)

    lanes = lax.iota(jnp.int32, L)

    def start_a(c, b):
        pltpu.async_copy(hp_hbm.at[idxi_v.at[c]], rows_v.at[b], sems.at[b])

    def start_b(c, b):
        pltpu.async_copy(hn_hbm.at[idxo_v.at[c]], rows_v.at[b], sems.at[b],
                         add=True)

    def drain(b):
        pltpu.make_async_copy(
            hp_hbm.at[idxi_v.at[0]], rows_v.at[b], sems.at[b]).wait()

    def compute(c, rows):
        cbase = jnp.full((L,), c * EPC, jnp.int32) + lanes
        for g in range(EPC // GRP):
            for l in range(GRP):
                m = g * GRP + l
                t = rows[m, pl.ds(0, 2 * L)]
                acc = t * t
                for d in range(1, D_FEAT // (2 * L)):
                    t = rows[m, pl.ds(2 * d * L, 2 * L)]
                    acc = acc + t * t
                a0, a1 = plsc.unpack(acc, format=plsc.PackFormat.INTERLEAVED)
                scr_v[l] = a0 + a1
            cols = [plsc.load_gather(
                        scr_v, [lanes, jnp.full((L,), k, jnp.int32)])
                    for k in range(L)]
            while len(cols) > 1:
                cols = [a + b for a, b in zip(cols[::2], cols[1::2])]
            plsc.store_scatter(sq_v, [cbase + (g * GRP)], cols[0])

    half = NBUF // 2
    for i in range(half):
        start_a(i, i)
    for i in range(half):
        drain(i)
        start_b(i, i)
        start_a(half + i, half + i)

    def ring_body(c, carry):
        b = lax.bitwise_and(c, NBUF - 1)
        drain(b)
        compute(c, rows_v.at[b])

        @pl.when(c + NBUF < NCH)
        def _():
            start_a(c + NBUF, b)

        j = lax.bitwise_and(c + half, NBUF - 1)

        @pl.when(c + half < NCH)
        def _():
            drain(j)
            start_b(c + half, j)

        return carry

    lax.fori_loop(0, NCH, ring_body, 0)

    inv_t = 1.0 / T_DEC

    def sig_body(t, carry):
        v = sq_v[pl.ds(t * L, L)]
        sq_v[pl.ds(t * L, L)] = 1.0 / (jnp.exp((v - R_DEC) * inv_t) + 1.0)
        return carry

    lax.fori_loop(0, EPW // L, sig_body, 0)
    pltpu.sync_copy(sq_v, out_hbm.at[pl.ds(ebase, EPW)])


@jax.jit
def _lp_call(hp, hn, idxi, idxo):
    mesh = plsc.VectorSubcoreMesh(core_axis_name="c", subcore_axis_name="s")
    fn = functools.partial(
        pl.kernel,
        out_type=jax.ShapeDtypeStruct((N_EDGES,), jnp.float32),
        mesh=mesh,
        compiler_params=pltpu.CompilerParams(needs_layout_passes=False,
                                             use_tc_tiling_on_sc=False),
        scratch_types=[
            pltpu.VMEM((NCH, EPC), jnp.int32),
            pltpu.VMEM((NCH, EPC), jnp.int32),
            pltpu.VMEM((NBUF, EPC, D_FEAT), jnp.bfloat16),
            pltpu.VMEM((EPW,), jnp.float32),
            pltpu.VMEM((L, L), jnp.float32),
            pltpu.SemaphoreType.DMA((NBUF,)),
        ],
    )(_lp_body)
    return fn(hp, hn, idxi, idxo)


def kernel(h, idx):
    hb = h.astype(jnp.bfloat16)
    idxi = idx[:, 0].astype(jnp.int32).reshape(NW, NCH, EPC)
    idxo = idx[:, 1].astype(jnp.int32).reshape(NW, NCH, EPC)
    return _lp_call(hb, -hb, idxi, idxo)
